# Initial kernel scaffold; baseline (speedup 1.0000x reference)
import jax
import jax.numpy as jnp
from jax.experimental import pallas as pl


def kernel(input_ids, embed_weight, weighted_sum, ln_gamma, ln_beta):
    raise NotImplementedError("write your pallas kernel here")


# speedup vs baseline: 4.8534x; 4.8534x over previous
import functools

import jax
import jax.numpy as jnp
from jax import lax
from jax.experimental import pallas as pl
from jax.experimental.pallas import tpu as pltpu
from jax.experimental.pallas import tpu_sc as plsc

_EMBED_SCALE = 1.0
_LN_EPS = 1e-5
_NC, _NS, _L = 2, 16, 16
_NW = _NC * _NS


def _ln_body(w_ref, g_ref, b_ref, o_ref):
    x = w_ref[...] * _EMBED_SCALE
    mean = jnp.mean(x, axis=-1, keepdims=True)
    xc = x - mean
    var = jnp.mean(xc * xc, axis=-1, keepdims=True)
    o_ref[...] = xc * lax.rsqrt(var + _LN_EPS) * g_ref[...] + b_ref[...]


def _ln_table(w, gamma, beta):
    v, e = w.shape
    blk = 1024
    return pl.pallas_call(
        _ln_body,
        grid=(v // blk,),
        in_specs=[
            pl.BlockSpec((blk, e), lambda i: (i, 0)),
            pl.BlockSpec((1, e), lambda i: (0, 0)),
            pl.BlockSpec((1, e), lambda i: (0, 0)),
        ],
        out_specs=pl.BlockSpec((blk, e), lambda i: (i, 0)),
        out_shape=jax.ShapeDtypeStruct((v, e), jnp.float32),
    )(w, gamma.reshape(1, e), beta.reshape(1, e))


@functools.lru_cache(maxsize=None)
def _make_sc_gather(e, n_out, cs):
    k_per_w = n_out // _NW
    r = 16
    n_chunk = k_per_w // r
    mesh = plsc.VectorSubcoreMesh(core_axis_name="c", subcore_axis_name="s")

    @functools.partial(
        pl.kernel,
        mesh=mesh,
        out_type=jax.ShapeDtypeStruct((n_out, e), jnp.float32),
        scratch_types=[
            pltpu.VMEM((r * cs,), jnp.int32),
            pltpu.VMEM((r * cs, e), jnp.float32),
            pltpu.VMEM((r, e), jnp.float32),
            pltpu.SemaphoreType.DMA,
        ],
    )
    def sc_gather(tab_hbm, idx_hbm, out_hbm, idx_v, rows_v, acc_v, sem):
        wid = lax.axis_index("s") * _NC + lax.axis_index("c")
        base = wid * k_per_w

        def chunk(i, carry):
            r0 = base + i * r
            pltpu.sync_copy(idx_hbm.at[pl.ds(r0 * cs, r * cs)], idx_v)
            pltpu.async_copy(tab_hbm.at[idx_v], rows_v, sem).wait()

            def row(rr, c2):
                for j in range(e // _L):
                    o = j * _L
                    acc_v[rr, pl.ds(o, _L)] = (
                        rows_v[cs * rr, pl.ds(o, _L)]
                        + rows_v[cs * rr + 1, pl.ds(o, _L)]
                        + rows_v[cs * rr + 2, pl.ds(o, _L)]
                        + rows_v[cs * rr + 3, pl.ds(o, _L)]
                    )
                return c2

            lax.fori_loop(0, r, row, 0)
            pltpu.sync_copy(acc_v, out_hbm.at[pl.ds(r0, r)])
            return carry

        lax.fori_loop(0, n_chunk, chunk, 0)

    return sc_gather


def kernel(input_ids, embed_weight, weighted_sum, ln_gamma, ln_beta):
    del weighted_sum
    cs = 4
    b, f = input_ids.shape
    code_dim = f // cs
    v, e = embed_weight.shape
    n_out = b * code_dim

    tab = _ln_table(embed_weight, ln_gamma, ln_beta)
    idx = jnp.transpose(input_ids.reshape(b, code_dim, cs), (1, 0, 2)).reshape(-1)
    out_flat = _make_sc_gather(e, n_out, cs)(tab, idx)
    return out_flat.reshape(b, code_dim, e)


# speedup vs baseline: 6.9598x; 1.4340x over previous
import functools

import jax
import jax.numpy as jnp
from jax import lax
from jax.experimental import pallas as pl
from jax.experimental.pallas import tpu as pltpu
from jax.experimental.pallas import tpu_sc as plsc

_EMBED_SCALE = 1.0
_LN_EPS = 1e-5
_NC, _NS, _L = 2, 16, 16
_NW = _NC * _NS


def _ln_body(w_ref, g_ref, b_ref, o_ref):
    x = w_ref[...] * _EMBED_SCALE
    mean = jnp.mean(x, axis=-1, keepdims=True)
    xc = x - mean
    var = jnp.mean(xc * xc, axis=-1, keepdims=True)
    o_ref[...] = xc * lax.rsqrt(var + _LN_EPS) * g_ref[...] + b_ref[...]


def _ln_table(w, gamma, beta):
    v, e = w.shape
    blk = 1024
    return pl.pallas_call(
        _ln_body,
        grid=(v // blk,),
        in_specs=[
            pl.BlockSpec((blk, e), lambda i: (i, 0)),
            pl.BlockSpec((1, e), lambda i: (0, 0)),
            pl.BlockSpec((1, e), lambda i: (0, 0)),
        ],
        out_specs=pl.BlockSpec((blk, e), lambda i: (i, 0)),
        out_shape=jax.ShapeDtypeStruct((v, e), jnp.float32),
    )(w, gamma.reshape(1, e), beta.reshape(1, e))


@functools.lru_cache(maxsize=None)
def _make_sc_gather(e, n_out, cs):
    k_per_w = n_out // _NW
    r = 16
    n_chunk = k_per_w // r
    mesh = plsc.VectorSubcoreMesh(core_axis_name="c", subcore_axis_name="s")

    @functools.partial(
        pl.kernel,
        mesh=mesh,
        out_type=jax.ShapeDtypeStruct((n_out, e), jnp.float32),
        scratch_types=[
            pltpu.VMEM((k_per_w * cs,), jnp.int32),
            pltpu.VMEM((2, r * cs, e), jnp.float32),
            pltpu.VMEM((2, r, e), jnp.float32),
            pltpu.SemaphoreType.DMA,
            pltpu.SemaphoreType.DMA,
            pltpu.SemaphoreType.DMA,
            pltpu.SemaphoreType.DMA,
        ],
    )
    def sc_gather(tab_hbm, idx_hbm, out_hbm, idx_v, rows_v, acc_v,
                  gsem0, gsem1, osem0, osem1):
        wid = lax.axis_index("s") * _NC + lax.axis_index("c")
        base = wid * k_per_w
        gsems = (gsem0, gsem1)
        osems = (osem0, osem1)

        pltpu.sync_copy(idx_hbm.at[pl.ds(base * cs, k_per_w * cs)], idx_v)

        def gather_cp(slot, i):
            return pltpu.make_async_copy(
                tab_hbm.at[idx_v.at[pl.ds(i * r * cs, r * cs)]],
                rows_v.at[slot], gsems[slot])

        def out_cp(slot, i):
            return pltpu.make_async_copy(
                acc_v.at[slot], out_hbm.at[pl.ds(base + i * r, r)],
                osems[slot])

        def accumulate(slot):
            def row(rr, c2):
                for j in range(e // _L):
                    o = j * _L
                    acc_v[slot, rr, pl.ds(o, _L)] = (
                        rows_v[slot, cs * rr, pl.ds(o, _L)]
                        + rows_v[slot, cs * rr + 1, pl.ds(o, _L)]
                        + rows_v[slot, cs * rr + 2, pl.ds(o, _L)]
                        + rows_v[slot, cs * rr + 3, pl.ds(o, _L)]
                    )
                return c2
            lax.fori_loop(0, r, row, 0)

        gather_cp(0, 0).start()

        def outer(i2, carry):
            for slot in range(2):
                i = i2 * 2 + slot

                @pl.when(i + 1 < n_chunk)
                def _():
                    gather_cp(1 - slot, i + 1).start()

                gather_cp(slot, i).wait()

                @pl.when(i >= 2)
                def _():
                    out_cp(slot, i - 2).wait()

                accumulate(slot)
                out_cp(slot, i).start()
            return carry

        lax.fori_loop(0, n_chunk // 2, outer, 0)
        out_cp(0, n_chunk - 2).wait()
        out_cp(1, n_chunk - 1).wait()

    return sc_gather


def kernel(input_ids, embed_weight, weighted_sum, ln_gamma, ln_beta):
    del weighted_sum
    cs = 4
    b, f = input_ids.shape
    code_dim = f // cs
    v, e = embed_weight.shape
    n_out = b * code_dim

    tab = _ln_table(embed_weight, ln_gamma, ln_beta)
    idx = jnp.transpose(input_ids.reshape(b, code_dim, cs), (1, 0, 2)).reshape(-1)
    out_flat = _make_sc_gather(e, n_out, cs)(tab, idx)
    return out_flat.reshape(b, code_dim, e)
